# Initial kernel scaffold; baseline (speedup 1.0000x reference)
import jax
import jax.numpy as jnp
from jax.experimental import pallas as pl


def kernel(x, edge_index, edge_attr, Wx, bx, We, be, Wm, bm, beta):
    raise NotImplementedError("write your pallas kernel here")


# speedup vs baseline: 2.7278x; 2.7278x over previous
import functools

import jax
import jax.numpy as jnp
from jax import lax
from jax.experimental import pallas as pl
from jax.experimental.pallas import tpu as pltpu
from jax.experimental.pallas import tpu_sc as plsc

N_NODES = 10000
N_EDGES = 320000
D = 128
D_EDGE = 16

NC = 2
NS = 16
NW = NC * NS
CH = 80
EDGES_PER_W = N_EDGES // NW
CHUNKS = EDGES_PER_W // CH
N_PAD = 10240
ROWS_PER_TEC = N_PAD // NS
FULL_FLUSHES = ROWS_PER_TEC // CH

_GB = 2000
_EB = 8000
_FB = 2000


def _g_body(x_ref, wx_ref, bx_ref, wm1_ref, g_ref):
    h = jnp.dot(x_ref[...], wx_ref[...], preferred_element_type=jnp.float32)
    h = h + bx_ref[...]
    g_ref[...] = jnp.dot(h, wm1_ref[...], preferred_element_type=jnp.float32)


def _compute_g(x, Wx, bx2, Wm1):
    return pl.pallas_call(
        _g_body,
        grid=(N_NODES // _GB,),
        in_specs=[
            pl.BlockSpec((_GB, D), lambda i: (i, 0)),
            pl.BlockSpec((D, D), lambda i: (0, 0)),
            pl.BlockSpec((1, D), lambda i: (0, 0)),
            pl.BlockSpec((D, D), lambda i: (0, 0)),
        ],
        out_specs=pl.BlockSpec((_GB, D), lambda i: (i, 0)),
        out_shape=jax.ShapeDtypeStruct((N_NODES, D), jnp.float32),
    )(x, Wx, bx2, Wm1)


def _p_body(e_ref, we_ref, be_ref, wm2_ref, bm_ref, p_ref):
    w2 = jnp.dot(we_ref[...], wm2_ref[...], preferred_element_type=jnp.float32)
    c2 = jnp.dot(be_ref[...], wm2_ref[...], preferred_element_type=jnp.float32)
    c2 = c2 + bm_ref[...]
    p = jnp.dot(e_ref[...], w2, preferred_element_type=jnp.float32)
    p_ref[...] = p + c2


def _compute_p(edge_attr, We, be2, Wm2, bm2):
    return pl.pallas_call(
        _p_body,
        grid=(N_EDGES // _EB,),
        in_specs=[
            pl.BlockSpec((_EB, D_EDGE), lambda i: (i, 0)),
            pl.BlockSpec((D_EDGE, D), lambda i: (0, 0)),
            pl.BlockSpec((1, D), lambda i: (0, 0)),
            pl.BlockSpec((D, D), lambda i: (0, 0)),
            pl.BlockSpec((1, D), lambda i: (0, 0)),
        ],
        out_specs=pl.BlockSpec((_EB, D), lambda i: (i, 0)),
        out_shape=jax.ShapeDtypeStruct((N_EDGES, D), jnp.float32),
    )(edge_attr, We, be2, Wm2, bm2)


_MESH = plsc.VectorSubcoreMesh(
    core_axis_name="c", subcore_axis_name="s", num_cores=NC, num_subcores=NS
)


@functools.partial(
    pl.kernel,
    out_type=jax.ShapeDtypeStruct((NC * N_PAD, D), jnp.float32),
    mesh=_MESH,
    scratch_types=[
        pltpu.VMEM((CH,), jnp.int32),
        pltpu.VMEM((CH,), jnp.int32),
        pltpu.VMEM((CH, D), jnp.float32),
        pltpu.VMEM((CH, D), jnp.float32),
        pltpu.VMEM_SHARED((N_PAD, D), jnp.float32),
        pltpu.SemaphoreType.DMA,
    ],
)
def _sc_aggregate(g_hbm, p_hbm, src_hbm, dst_hbm, out_hbm,
                  sidx, didx, gbuf, pbuf, acc, sem):
    cid = lax.axis_index("c")
    sid = lax.axis_index("s")
    wid = sid * NC + cid
    base = sid * ROWS_PER_TEC

    zero = jnp.zeros((16,), jnp.float32)

    def _zrow(r, carry):
        for c in range(D // 16):
            gbuf[r, pl.ds(c * 16, 16)] = zero
        return carry

    lax.fori_loop(0, CH, _zrow, 0)

    def _zcopy(i, carry):
        pltpu.sync_copy(gbuf, acc.at[pl.ds(base + i * CH, CH)])
        return carry

    lax.fori_loop(0, FULL_FLUSHES, _zcopy, 0)
    plsc.subcore_barrier()

    def _chunk(k, carry):
        ebase = wid * EDGES_PER_W + k * CH
        pltpu.sync_copy(src_hbm.at[pl.ds(ebase, CH)], sidx)
        pltpu.sync_copy(dst_hbm.at[pl.ds(ebase, CH)], didx)
        pltpu.async_copy(g_hbm.at[sidx], gbuf, sem).wait()
        pltpu.sync_copy(p_hbm.at[pl.ds(ebase, CH)], pbuf)

        def _row(r, rc):
            for c in range(D // 16):
                sl = pl.ds(c * 16, 16)
                m = gbuf[r, sl] + pbuf[r, sl]
                pbuf[r, sl] = jnp.maximum(m, m * jnp.float32(0.01))
            return rc

        lax.fori_loop(0, CH, _row, 0)
        pltpu.sync_copy(pbuf, acc.at[didx], add=True)
        return carry

    lax.fori_loop(0, CHUNKS, _chunk, 0)
    plsc.subcore_barrier()

    obase = cid * N_PAD + base

    def _flush(i, carry):
        pltpu.sync_copy(acc.at[pl.ds(base + i * CH, CH)], gbuf)
        pltpu.sync_copy(gbuf, out_hbm.at[pl.ds(obase + i * CH, CH)])
        return carry

    lax.fori_loop(0, FULL_FLUSHES, _flush, 0)


def _f_body(p_ref, beta_ref, o_ref):
    s = p_ref[0] + p_ref[1]
    b = jnp.maximum(beta_ref[0, 0], jnp.float32(0.0))
    o_ref[...] = jax.nn.sigmoid(s) * b


def _finalize(parts, beta2):
    return pl.pallas_call(
        _f_body,
        grid=(N_NODES // _FB,),
        in_specs=[
            pl.BlockSpec((NC, _FB, D), lambda i: (0, i, 0)),
            pl.BlockSpec(memory_space=pltpu.SMEM),
        ],
        out_specs=pl.BlockSpec((_FB, D), lambda i: (i, 0)),
        out_shape=jax.ShapeDtypeStruct((N_NODES, D), jnp.float32),
    )(parts, beta2)


def kernel(x, edge_index, edge_attr, Wx, bx, We, be, Wm, bm, beta):
    Wm1 = Wm[:D]
    Wm2 = Wm[D:]
    src = edge_index[0]
    dst = edge_index[1]
    g = _compute_g(x, Wx, bx.reshape(1, D), Wm1)
    p = _compute_p(edge_attr, We, be.reshape(1, D), Wm2, bm.reshape(1, D))
    parts = _sc_aggregate(g, p, src, dst)
    parts = parts.reshape(NC, N_PAD, D)
    return _finalize(parts, beta.reshape(1, 1))
